# Initial kernel scaffold; baseline (speedup 1.0000x reference)
import jax
import jax.numpy as jnp
from jax.experimental import pallas as pl


def kernel(x, edge_index, edge_attr, Wx, bx, We, be):
    raise NotImplementedError("write your pallas kernel here")


# speedup vs baseline: 1.5788x; 1.5788x over previous
import functools

import jax
import jax.numpy as jnp
from jax import lax
from jax.experimental import pallas as pl
from jax.experimental.pallas import tpu as pltpu
from jax.experimental.pallas import tpu_sc as plsc

N_NODES = 10000
E_EDGES = 160000
D_FEAT = 256
D_EDGE = 16


def _node_matmul_body(x_ref, w_ref, b_ref, o_ref):
    acc = lax.dot_general(x_ref[...], w_ref[...], (((1,), (1,)), ((), ())),
                          preferred_element_type=jnp.float32)
    o_ref[...] = acc + 0.5 * b_ref[...]


def _node_matmul(x, Wx, bx):
    blk = 1000
    return pl.pallas_call(
        _node_matmul_body,
        grid=(N_NODES // blk,),
        in_specs=[
            pl.BlockSpec((blk, D_FEAT), lambda i: (i, 0)),
            pl.BlockSpec((D_FEAT, D_FEAT), lambda i: (0, 0)),
            pl.BlockSpec((1, D_FEAT), lambda i: (0, 0)),
        ],
        out_specs=pl.BlockSpec((blk, D_FEAT), lambda i: (i, 0)),
        out_shape=jax.ShapeDtypeStruct((N_NODES, D_FEAT), jnp.float32),
    )(x, Wx, bx.reshape(1, D_FEAT))


def _edge_matmul_body(a_ref, w_ref, b_ref, o_ref):
    acc = lax.dot_general(a_ref[...], w_ref[...], (((1,), (1,)), ((), ())),
                          preferred_element_type=jnp.float32)
    o_ref[...] = acc + b_ref[...]


def _edge_matmul(edge_attr, We, be):
    blk = 8000
    return pl.pallas_call(
        _edge_matmul_body,
        grid=(E_EDGES // blk,),
        in_specs=[
            pl.BlockSpec((blk, D_EDGE), lambda i: (i, 0)),
            pl.BlockSpec((D_EDGE, D_EDGE), lambda i: (0, 0)),
            pl.BlockSpec((1, D_EDGE), lambda i: (0, 0)),
        ],
        out_specs=pl.BlockSpec((blk, D_EDGE), lambda i: (i, 0)),
        out_shape=jax.ShapeDtypeStruct((E_EDGES, D_EDGE), jnp.float32),
    )(edge_attr, We, be.reshape(1, D_EDGE))


_NC, _NS, _LANES = 2, 16, 16
_NW = _NC * _NS
_EPW = E_EDGES // _NW
_CHUNK = 200
_NCHUNK = _EPW // _CHUNK


def _sc_gather_sum(y, idx_i, idx_j):
    mesh = plsc.VectorSubcoreMesh(core_axis_name="c", subcore_axis_name="s")

    @functools.partial(
        pl.kernel,
        mesh=mesh,
        out_type=jax.ShapeDtypeStruct((E_EDGES, D_FEAT), jnp.float32),
        scratch_types=[
            pltpu.VMEM((_EPW,), jnp.int32),
            pltpu.VMEM((_EPW,), jnp.int32),
            pltpu.VMEM((_CHUNK, D_FEAT), jnp.float32),
            pltpu.VMEM((_CHUNK, D_FEAT), jnp.float32),
            pltpu.SemaphoreType.DMA,
            pltpu.SemaphoreType.DMA,
        ],
    )
    def body(y_hbm, ii_hbm, jj_hbm, out_hbm, ii_v, jj_v, a_v, b_v, sema, semb):
        wid = lax.axis_index("s") * _NC + lax.axis_index("c")
        base = wid * _EPW
        pltpu.sync_copy(ii_hbm.at[pl.ds(base, _EPW)], ii_v)
        pltpu.sync_copy(jj_hbm.at[pl.ds(base, _EPW)], jj_v)

        def chunk_body(c, carry):
            off = c * _CHUNK
            cpa = pltpu.async_copy(y_hbm.at[ii_v.at[pl.ds(off, _CHUNK)]], a_v, sema)
            cpb = pltpu.async_copy(y_hbm.at[jj_v.at[pl.ds(off, _CHUNK)]], b_v, semb)
            cpa.wait()
            cpb.wait()

            def row_body(r, rcarry):
                for k in range(D_FEAT // _LANES):
                    sl = pl.ds(k * _LANES, _LANES)
                    a_v[r, sl] = a_v[r, sl] + b_v[r, sl]
                return rcarry

            lax.fori_loop(0, _CHUNK, row_body, 0)
            pltpu.sync_copy(a_v, out_hbm.at[pl.ds(base + off, _CHUNK)])
            return carry

        lax.fori_loop(0, _NCHUNK, chunk_body, 0)

    return body(y, idx_i, idx_j)


def kernel(x, edge_index, edge_attr, Wx, bx, We, be):
    ei = edge_index.astype(jnp.int32)
    y = _node_matmul(x, Wx, bx)
    h = _sc_gather_sum(y, ei[0], ei[1])
    e = _edge_matmul(edge_attr, We, be)
    return jnp.concatenate((h, e), axis=1)
